# baseline (speedup 1.0000x reference)
import functools
import jax
import jax.numpy as jnp
from jax import lax
from jax.experimental import pallas as pl
from jax.experimental.pallas import tpu as pltpu
from jax.experimental.pallas import tpu_sc as plsc

N_NODES = 10000
N_EDGES = 320000
F_IN = 128
HID = 16
N_CLS = 100

NC = 2
NS = 16
NW = NC * NS
B_EDGE = 128
CH = 80
G_EDGE = 1024
CHG = (CH * B_EDGE) // G_EDGE
SPG = G_EDGE // B_EDGE
E_PAD = NW * CH * B_EDGE
CHD = 2 * CH
CHG0 = 16
CHG1 = 2 * CHG - CHG0
CH0 = CHG0 * SPG
CH1 = CHD - CH0
CHGMX = max(CHG0, CHG1)
NPAD = 10240
RPT = NPAD // NS
RPT_LAST = N_NODES - (NS - 1) * RPT

_mesh = plsc.VectorSubcoreMesh(
    core_axis_name="c", subcore_axis_name="s", num_cores=NC, num_subcores=NS
)
_sc_params = pltpu.CompilerParams(
    use_tc_tiling_on_sc=False, needs_layout_passes=False
)

_f32 = jnp.float32
_i32 = jnp.int32


def _bcast_row(vref, i):
    return plsc.load_gather(vref, [jnp.full((16,), i, _i32)])


def _row(ref2d, i):
    return plsc.load_gather(ref2d, [jnp.full((16,), i, _i32), lax.iota(_i32, 16)])


def _zero_vmem_rows(ref2d, n):
    for i in range(n):
        ref2d[i] = jnp.zeros((16,), _f32)


def _newton_rsqrt(x):
    i = plsc.bitcast(x, _i32)
    i = jnp.int32(0x5F3759DF) - lax.shift_right_arithmetic(i, 1)
    y = plsc.bitcast(i, _f32)
    for _ in range(3):
        y = y * (1.5 - 0.5 * x * y * y)
    return y


def _agg_pipeline(t_hbm, src_v, dd_v, rows0_v, rows1_v, acc_sh, base, chg,
                  sg0, sg1, ss0, ss1):
    def _scat_start(rows_v, chunk, sem):
        pltpu.async_copy(
            rows_v.at[pl.ds((chunk % SPG) * B_EDGE, B_EDGE)],
            acc_sh.at[dd_v.at[base + chunk]],
            sem,
            add=True,
        )

    def _scat(rows_v, chunk, sem):
        return pltpu.make_async_copy(
            rows_v.at[pl.ds((chunk % SPG) * B_EDGE, B_EDGE)],
            acc_sh.at[dd_v.at[base + chunk]],
            sem,
        )

    pltpu.async_copy(t_hbm.at[src_v.at[0]], rows0_v, sg0)

    def pair(i, carry):
        j0 = 2 * i
        j1 = j0 + 1
        pltpu.make_async_copy(t_hbm.at[src_v.at[j0]], rows0_v, sg0).wait()

        @pl.when(i > 0)
        def _():
            for k in range(SPG):
                _scat(rows1_v, SPG * (j0 - 1) + k, ss1).wait()

        pltpu.async_copy(t_hbm.at[src_v.at[j1]], rows1_v, sg1)
        for k in range(SPG):
            _scat_start(rows0_v, SPG * j0 + k, ss0)
        pltpu.make_async_copy(t_hbm.at[src_v.at[j1]], rows1_v, sg1).wait()
        for k in range(SPG):
            _scat(rows0_v, SPG * j0 + k, ss0).wait()

        @pl.when(i < chg // 2 - 1)
        def _():
            pltpu.async_copy(t_hbm.at[src_v.at[j0 + 2]], rows0_v, sg0)

        for k in range(SPG):
            _scat_start(rows1_v, SPG * j1 + k, ss1)
        return carry

    lax.fori_loop(0, chg // 2, pair, 0)
    for k in range(SPG):
        _scat(rows1_v, SPG * (chg - 1) + k, ss1).wait()


def _split_agg(t_hbm, src_v, dd_v, rows0_v, rows1_v, acc_sh, cid,
               sg0, sg1, ss0, ss1):
    @pl.when(cid == 0)
    def _():
        _agg_pipeline(t_hbm, src_v, dd_v, rows0_v, rows1_v, acc_sh, 0, CHG0,
                      sg0, sg1, ss0, ss1)

    @pl.when(cid == 1)
    def _():
        _agg_pipeline(t_hbm, src_v, dd_v, rows0_v, rows1_v, acc_sh, CH0, CHG1,
                      sg0, sg1, ss0, ss1)


def _load_idx(dstf_hbm, srcf_hbm, dd_v, src_v, cid, sid):
    pltpu.sync_copy(dstf_hbm.at[sid], dd_v)

    @pl.when(cid == 0)
    def _():
        pltpu.sync_copy(srcf_hbm.at[sid].at[pl.ds(0, CHG0)],
                        src_v.at[pl.ds(0, CHG0)])

    @pl.when(cid == 1)
    def _():
        pltpu.sync_copy(srcf_hbm.at[sid].at[pl.ds(CHG0, CHG1)],
                        src_v.at[pl.ds(0, CHG1)])


@functools.partial(
    pl.kernel,
    out_type=(
        jax.ShapeDtypeStruct((NC, NPAD, HID), _f32),
        jax.ShapeDtypeStruct((NC, NPAD, HID), _f32),
        jax.ShapeDtypeStruct((NC, NPAD), _f32),
        jax.ShapeDtypeStruct((NPAD,), _f32),
    ),
    mesh=_mesh,
    compiler_params=_sc_params,
    scratch_types=[
        pltpu.VMEM((CHD, B_EDGE), _i32),
        pltpu.VMEM((CHGMX, G_EDGE), _i32),
        pltpu.VMEM((G_EDGE, HID), _f32),
        pltpu.VMEM((G_EDGE, HID), _f32),
        pltpu.VMEM((RPT, HID), _f32),
        pltpu.VMEM((RPT,), _f32),
        pltpu.VMEM((B_EDGE,), _f32),
        pltpu.VMEM_SHARED((NPAD, HID), _f32),
        pltpu.VMEM_SHARED((NPAD,), _f32),
        pltpu.SemaphoreType.DMA,
        pltpu.SemaphoreType.DMA,
        pltpu.SemaphoreType.DMA,
        pltpu.SemaphoreType.DMA,
        pltpu.SemaphoreType.DMA,
    ],
)
def _sc_a(hw1_hbm, dstf_hbm, srcf_hbm,
          out_p, out_t1, out_norm, out_normc,
          dd_v, src_v, rows0_v, rows1_v, t1_v, nv, ones_v,
          acc_sh, accd_sh, sg0, sg1, ss0, ss1, sh):
    cid = lax.axis_index("c")
    sid = lax.axis_index("s")

    @pl.when(sid < NS - 1)
    def _():
        pltpu.async_copy(hw1_hbm.at[pl.ds(sid * RPT, RPT)], t1_v, sh)

    @pl.when(sid == NS - 1)
    def _():
        pltpu.async_copy(
            hw1_hbm.at[pl.ds((NS - 1) * RPT, RPT_LAST)],
            t1_v.at[pl.ds(0, RPT_LAST)], sh,
        )
        for i in range(RPT_LAST, RPT):
            t1_v[i] = jnp.zeros((16,), _f32)

    _load_idx(dstf_hbm, srcf_hbm, dd_v, src_v, cid, sid)

    for i in range(RPT // 16):
        nv[pl.ds(i * 16, 16)] = jnp.zeros((16,), _f32)
    pltpu.sync_copy(nv, accd_sh.at[pl.ds(sid * RPT, RPT)])
    _zero_vmem_rows(rows0_v, B_EDGE)
    for k in range(RPT // B_EDGE):
        pltpu.sync_copy(
            rows0_v.at[pl.ds(0, B_EDGE)],
            acc_sh.at[pl.ds(sid * RPT + k * B_EDGE, B_EDGE)],
        )
    for i in range(B_EDGE // 16):
        ones_v[pl.ds(i * 16, 16)] = jnp.ones((16,), _f32)
    plsc.subcore_barrier()

    def deg_body(i, carry):
        for k in range(8):
            pltpu.async_copy(
                ones_v, accd_sh.at[dd_v.at[8 * i + k]], ss0, add=True
            )
        for k in range(8):
            pltpu.make_async_copy(
                ones_v, accd_sh.at[dd_v.at[8 * i + k]], ss0
            ).wait()
        return carry

    lax.fori_loop(0, CHD // 8, deg_body, 0)
    plsc.subcore_barrier()

    pltpu.sync_copy(accd_sh.at[pl.ds(sid * RPT, RPT)], nv)
    for i in range(RPT // 16):
        d = nv[pl.ds(i * 16, 16)]
        nv[pl.ds(i * 16, 16)] = _newton_rsqrt(d + 1.0)
    pltpu.sync_copy(nv, out_norm.at[cid].at[pl.ds(sid * RPT, RPT)])

    @pl.when(cid == 0)
    def _():
        pltpu.sync_copy(nv, out_normc.at[pl.ds(sid * RPT, RPT)])

    @pl.when(sid < NS - 1)
    def _():
        pltpu.make_async_copy(
            hw1_hbm.at[pl.ds(sid * RPT, RPT)], t1_v, sh
        ).wait()

    @pl.when(sid == NS - 1)
    def _():
        pltpu.make_async_copy(
            hw1_hbm.at[pl.ds((NS - 1) * RPT, RPT_LAST)],
            t1_v.at[pl.ds(0, RPT_LAST)], sh,
        ).wait()

    def scale_body(i, carry):
        t1_v[i] = t1_v[i] * _bcast_row(nv, i)
        return carry

    lax.fori_loop(0, RPT, scale_body, 0)
    pltpu.sync_copy(t1_v, out_t1.at[cid].at[pl.ds(sid * RPT, RPT)])
    plsc.subcore_barrier()

    _split_agg(out_t1.at[cid], src_v, dd_v, rows0_v, rows1_v, acc_sh, cid,
               sg0, sg1, ss0, ss1)
    plsc.subcore_barrier()
    pltpu.sync_copy(
        acc_sh.at[pl.ds(sid * RPT, RPT)],
        out_p.at[cid].at[pl.ds(sid * RPT, RPT)],
    )


@functools.partial(
    pl.kernel,
    out_type=(
        jax.ShapeDtypeStruct((NC, NPAD, HID), _f32),
        jax.ShapeDtypeStruct((NC, NPAD, HID), _f32),
    ),
    mesh=_mesh,
    compiler_params=_sc_params,
    scratch_types=[
        pltpu.VMEM((CHD, B_EDGE), _i32),
        pltpu.VMEM((CHGMX, G_EDGE), _i32),
        pltpu.VMEM((G_EDGE, HID), _f32),
        pltpu.VMEM((G_EDGE, HID), _f32),
        pltpu.VMEM((RPT, HID), _f32),
        pltpu.VMEM((RPT, HID), _f32),
        pltpu.VMEM((RPT, HID), _f32),
        pltpu.VMEM((RPT,), _f32),
        pltpu.VMEM((16,), _f32),
        pltpu.VMEM_SHARED((NPAD, HID), _f32),
        pltpu.SemaphoreType.DMA,
        pltpu.SemaphoreType.DMA,
        pltpu.SemaphoreType.DMA,
        pltpu.SemaphoreType.DMA,
        pltpu.SemaphoreType.DMA,
    ],
)
def _sc_b(p_hbm, t1_hbm, norm_hbm, b1_hbm, dstf_hbm, srcf_hbm,
          out_q, out_t2, dd_v, src_v, rows0_v, rows1_v, t2_v, p1_v, t1_v, nv,
          b1_v, acc_sh, sg0, sg1, ss0, ss1, sh):
    cid = lax.axis_index("c")
    sid = lax.axis_index("s")

    pltpu.async_copy(p_hbm.at[0].at[pl.ds(sid * RPT, RPT)], t2_v, sh)
    _load_idx(dstf_hbm, srcf_hbm, dd_v, src_v, cid, sid)
    pltpu.sync_copy(p_hbm.at[1].at[pl.ds(sid * RPT, RPT)], p1_v)
    pltpu.sync_copy(t1_hbm.at[cid].at[pl.ds(sid * RPT, RPT)], t1_v)
    pltpu.sync_copy(norm_hbm.at[cid].at[pl.ds(sid * RPT, RPT)], nv)
    pltpu.sync_copy(b1_hbm, b1_v)
    _zero_vmem_rows(rows0_v, B_EDGE)
    for k in range(RPT // B_EDGE):
        pltpu.sync_copy(
            rows0_v.at[pl.ds(0, B_EDGE)],
            acc_sh.at[pl.ds(sid * RPT + k * B_EDGE, B_EDGE)],
        )
    pltpu.make_async_copy(p_hbm.at[0].at[pl.ds(sid * RPT, RPT)], t2_v, sh).wait()
    b1r = b1_v[...]

    def relu_body(i, carry):
        nb = _bcast_row(nv, i)
        agg = t2_v[i] + p1_v[i] + t1_v[i]
        h = jnp.maximum(agg * nb + b1r, 0.0)
        t2_v[i] = h * nb
        return carry

    lax.fori_loop(0, RPT, relu_body, 0)
    pltpu.sync_copy(t2_v, out_t2.at[cid].at[pl.ds(sid * RPT, RPT)])
    plsc.subcore_barrier()

    _split_agg(out_t2.at[cid], src_v, dd_v, rows0_v, rows1_v, acc_sh, cid,
               sg0, sg1, ss0, ss1)
    plsc.subcore_barrier()
    pltpu.sync_copy(
        acc_sh.at[pl.ds(sid * RPT, RPT)],
        out_q.at[cid].at[pl.ds(sid * RPT, RPT)],
    )


_RB = 1280
_RB2 = 1000


def _mm1_body(x_ref, w1_ref, o_ref):
    o_ref[...] = jnp.dot(x_ref[...], w1_ref[...], preferred_element_type=_f32)


def _post_body(qa_ref, qb_ref, t2_ref, norm_ref, w2_ref, b2_ref, o_ref):
    g = (qa_ref[0] + qb_ref[0] + t2_ref[0]) * norm_ref[...]
    z = jnp.dot(g, w2_ref[...], preferred_element_type=_f32) + b2_ref[...]
    m = jnp.max(z, axis=1, keepdims=True)
    e = z - m
    lse = jnp.log(jnp.sum(jnp.exp(e), axis=1, keepdims=True))
    o_ref[...] = e - lse


def _row_spec(c):
    return pl.BlockSpec((_RB, c), lambda i: (i, 0))


def _full_spec(r, c):
    return pl.BlockSpec((r, c), lambda i: (0, 0))


def kernel(x, edge_index, W1, b1, W2, b2):
    ep = jnp.pad(edge_index, ((0, 0), (0, E_PAD - N_EDGES)),
                 constant_values=N_NODES)
    srcf = ep[0].reshape(NS, 2 * CHG, G_EDGE)
    dstf = ep[1].reshape(NS, CHD, B_EDGE)
    b2r = b2.reshape(1, N_CLS)

    hw1 = pl.pallas_call(
        _mm1_body,
        grid=(N_NODES // _RB2,),
        in_specs=[pl.BlockSpec((_RB2, F_IN), lambda i: (i, 0)),
                  _full_spec(F_IN, HID)],
        out_specs=pl.BlockSpec((_RB2, HID), lambda i: (i, 0)),
        out_shape=jax.ShapeDtypeStruct((N_NODES, HID), _f32),
    )(x, W1)

    p, t1, norm, normc = _sc_a(hw1, dstf, srcf)

    q, t2 = _sc_b(p, t1, norm, b1, dstf, srcf)

    def _rs2(c):
        return pl.BlockSpec((_RB2, c), lambda i: (i, 0))

    def _rs3(lead):
        return pl.BlockSpec((1, _RB2, HID), lambda i, _l=lead: (_l, i, 0))

    out = pl.pallas_call(
        _post_body,
        grid=(N_NODES // _RB2,),
        in_specs=[_rs3(0), _rs3(1), _rs3(0), _rs2(1),
                  _full_spec(HID, N_CLS), _full_spec(1, N_CLS)],
        out_specs=_rs2(N_CLS),
        out_shape=jax.ShapeDtypeStruct((N_NODES, N_CLS), _f32),
    )(q, q, t2, normc.reshape(NPAD, 1), W2, b2r)

    return out


# speedup vs baseline: 1.0729x; 1.0729x over previous
import functools
import jax
import jax.numpy as jnp
from jax import lax
from jax.experimental import pallas as pl
from jax.experimental.pallas import tpu as pltpu
from jax.experimental.pallas import tpu_sc as plsc

N_NODES = 10000
N_EDGES = 320000
F_IN = 128
HID = 16
N_CLS = 100

NC = 2
NS = 16
NW = NC * NS
B_EDGE = 128
CH = 80
G_EDGE = 1024
CHG = (CH * B_EDGE) // G_EDGE
SPG = G_EDGE // B_EDGE
E_PAD = NW * CH * B_EDGE
CHD = 2 * CH
CHG0 = 14
CHG1 = 2 * CHG - CHG0
CH0 = CHG0 * SPG
CH1 = CHD - CH0
CHGMX = max(CHG0, CHG1)
NPAD = 10240
RPT = NPAD // NS
RPT_LAST = N_NODES - (NS - 1) * RPT

_mesh = plsc.VectorSubcoreMesh(
    core_axis_name="c", subcore_axis_name="s", num_cores=NC, num_subcores=NS
)
_sc_params = pltpu.CompilerParams(
    use_tc_tiling_on_sc=False, needs_layout_passes=False
)

_f32 = jnp.float32
_i32 = jnp.int32


def _bcast_row(vref, i):
    return plsc.load_gather(vref, [jnp.full((16,), i, _i32)])


def _row(ref2d, i):
    return plsc.load_gather(ref2d, [jnp.full((16,), i, _i32), lax.iota(_i32, 16)])


def _zero_vmem_rows(ref2d, n):
    for i in range(n):
        ref2d[i] = jnp.zeros((16,), _f32)


def _newton_rsqrt(x):
    i = plsc.bitcast(x, _i32)
    i = jnp.int32(0x5F3759DF) - lax.shift_right_arithmetic(i, 1)
    y = plsc.bitcast(i, _f32)
    for _ in range(3):
        y = y * (1.5 - 0.5 * x * y * y)
    return y


def _agg_pipeline(t_hbm, src_v, dd_v, rows0_v, rows1_v, acc_sh, base, chg,
                  sg0, sg1, ss0, ss1):
    def _scat_start(rows_v, chunk, sem):
        pltpu.async_copy(
            rows_v.at[pl.ds((chunk % SPG) * B_EDGE, B_EDGE)],
            acc_sh.at[dd_v.at[base + chunk]],
            sem,
            add=True,
        )

    def _scat(rows_v, chunk, sem):
        return pltpu.make_async_copy(
            rows_v.at[pl.ds((chunk % SPG) * B_EDGE, B_EDGE)],
            acc_sh.at[dd_v.at[base + chunk]],
            sem,
        )

    pltpu.async_copy(t_hbm.at[src_v.at[0]], rows0_v, sg0)

    def pair(i, carry):
        j0 = 2 * i
        j1 = j0 + 1
        pltpu.make_async_copy(t_hbm.at[src_v.at[j0]], rows0_v, sg0).wait()

        @pl.when(i > 0)
        def _():
            for k in range(SPG):
                _scat(rows1_v, SPG * (j0 - 1) + k, ss1).wait()

        pltpu.async_copy(t_hbm.at[src_v.at[j1]], rows1_v, sg1)
        for k in range(SPG):
            _scat_start(rows0_v, SPG * j0 + k, ss0)
        pltpu.make_async_copy(t_hbm.at[src_v.at[j1]], rows1_v, sg1).wait()
        for k in range(SPG):
            _scat(rows0_v, SPG * j0 + k, ss0).wait()

        @pl.when(i < chg // 2 - 1)
        def _():
            pltpu.async_copy(t_hbm.at[src_v.at[j0 + 2]], rows0_v, sg0)

        for k in range(SPG):
            _scat_start(rows1_v, SPG * j1 + k, ss1)
        return carry

    lax.fori_loop(0, chg // 2, pair, 0)
    for k in range(SPG):
        _scat(rows1_v, SPG * (chg - 1) + k, ss1).wait()


def _split_agg(t_hbm, src_v, dd_v, rows0_v, rows1_v, acc_sh, cid,
               sg0, sg1, ss0, ss1):
    @pl.when(cid == 0)
    def _():
        _agg_pipeline(t_hbm, src_v, dd_v, rows0_v, rows1_v, acc_sh, 0, CHG0,
                      sg0, sg1, ss0, ss1)

    @pl.when(cid == 1)
    def _():
        _agg_pipeline(t_hbm, src_v, dd_v, rows0_v, rows1_v, acc_sh, CH0, CHG1,
                      sg0, sg1, ss0, ss1)


def _load_idx(dstf_hbm, srcf_hbm, dd_v, src_v, cid, sid):
    pltpu.sync_copy(dstf_hbm.at[sid], dd_v)

    @pl.when(cid == 0)
    def _():
        pltpu.sync_copy(srcf_hbm.at[sid].at[pl.ds(0, CHG0)],
                        src_v.at[pl.ds(0, CHG0)])

    @pl.when(cid == 1)
    def _():
        pltpu.sync_copy(srcf_hbm.at[sid].at[pl.ds(CHG0, CHG1)],
                        src_v.at[pl.ds(0, CHG1)])


@functools.partial(
    pl.kernel,
    out_type=(
        jax.ShapeDtypeStruct((NC, NPAD, HID), _f32),
        jax.ShapeDtypeStruct((NC, NPAD, HID), _f32),
        jax.ShapeDtypeStruct((NC, NPAD), _f32),
        jax.ShapeDtypeStruct((NPAD,), _f32),
    ),
    mesh=_mesh,
    compiler_params=_sc_params,
    scratch_types=[
        pltpu.VMEM((CHD, B_EDGE), _i32),
        pltpu.VMEM((CHGMX, G_EDGE), _i32),
        pltpu.VMEM((G_EDGE, HID), _f32),
        pltpu.VMEM((G_EDGE, HID), _f32),
        pltpu.VMEM((RPT, HID), _f32),
        pltpu.VMEM((RPT,), _f32),
        pltpu.VMEM((B_EDGE,), _f32),
        pltpu.VMEM_SHARED((NPAD, HID), _f32),
        pltpu.VMEM_SHARED((NPAD,), _f32),
        pltpu.SemaphoreType.DMA,
        pltpu.SemaphoreType.DMA,
        pltpu.SemaphoreType.DMA,
        pltpu.SemaphoreType.DMA,
        pltpu.SemaphoreType.DMA,
    ],
)
def _sc_a(hw1_hbm, dstf_hbm, srcf_hbm,
          out_p, out_t1, out_norm, out_normc,
          dd_v, src_v, rows0_v, rows1_v, t1_v, nv, ones_v,
          acc_sh, accd_sh, sg0, sg1, ss0, ss1, sh):
    cid = lax.axis_index("c")
    sid = lax.axis_index("s")

    @pl.when(sid < NS - 1)
    def _():
        pltpu.async_copy(hw1_hbm.at[pl.ds(sid * RPT, RPT)], t1_v, sh)

    @pl.when(sid == NS - 1)
    def _():
        pltpu.async_copy(
            hw1_hbm.at[pl.ds((NS - 1) * RPT, RPT_LAST)],
            t1_v.at[pl.ds(0, RPT_LAST)], sh,
        )
        for i in range(RPT_LAST, RPT):
            t1_v[i] = jnp.zeros((16,), _f32)

    _load_idx(dstf_hbm, srcf_hbm, dd_v, src_v, cid, sid)

    for i in range(RPT // 16):
        nv[pl.ds(i * 16, 16)] = jnp.zeros((16,), _f32)
    pltpu.sync_copy(nv, accd_sh.at[pl.ds(sid * RPT, RPT)])
    _zero_vmem_rows(rows0_v, B_EDGE)
    for k in range(RPT // B_EDGE):
        pltpu.sync_copy(
            rows0_v.at[pl.ds(0, B_EDGE)],
            acc_sh.at[pl.ds(sid * RPT + k * B_EDGE, B_EDGE)],
        )
    for i in range(B_EDGE // 16):
        ones_v[pl.ds(i * 16, 16)] = jnp.ones((16,), _f32)
    plsc.subcore_barrier()

    def deg_body(i, carry):
        for k in range(8):
            pltpu.async_copy(
                ones_v, accd_sh.at[dd_v.at[8 * i + k]], ss0, add=True
            )
        for k in range(8):
            pltpu.make_async_copy(
                ones_v, accd_sh.at[dd_v.at[8 * i + k]], ss0
            ).wait()
        return carry

    lax.fori_loop(0, CHD // 8, deg_body, 0)
    plsc.subcore_barrier()

    pltpu.sync_copy(accd_sh.at[pl.ds(sid * RPT, RPT)], nv)
    for i in range(RPT // 16):
        d = nv[pl.ds(i * 16, 16)]
        nv[pl.ds(i * 16, 16)] = _newton_rsqrt(d + 1.0)
    pltpu.sync_copy(nv, out_norm.at[cid].at[pl.ds(sid * RPT, RPT)])

    @pl.when(cid == 0)
    def _():
        pltpu.sync_copy(nv, out_normc.at[pl.ds(sid * RPT, RPT)])

    @pl.when(sid < NS - 1)
    def _():
        pltpu.make_async_copy(
            hw1_hbm.at[pl.ds(sid * RPT, RPT)], t1_v, sh
        ).wait()

    @pl.when(sid == NS - 1)
    def _():
        pltpu.make_async_copy(
            hw1_hbm.at[pl.ds((NS - 1) * RPT, RPT_LAST)],
            t1_v.at[pl.ds(0, RPT_LAST)], sh,
        ).wait()

    def scale_body(i, carry):
        t1_v[i] = t1_v[i] * _bcast_row(nv, i)
        return carry

    lax.fori_loop(0, RPT, scale_body, 0)
    pltpu.sync_copy(t1_v, out_t1.at[cid].at[pl.ds(sid * RPT, RPT)])
    plsc.subcore_barrier()

    _split_agg(out_t1.at[cid], src_v, dd_v, rows0_v, rows1_v, acc_sh, cid,
               sg0, sg1, ss0, ss1)
    plsc.subcore_barrier()
    pltpu.sync_copy(
        acc_sh.at[pl.ds(sid * RPT, RPT)],
        out_p.at[cid].at[pl.ds(sid * RPT, RPT)],
    )


@functools.partial(
    pl.kernel,
    out_type=(
        jax.ShapeDtypeStruct((NC, NPAD, HID), _f32),
        jax.ShapeDtypeStruct((NC, NPAD, HID), _f32),
    ),
    mesh=_mesh,
    compiler_params=_sc_params,
    scratch_types=[
        pltpu.VMEM((CHD, B_EDGE), _i32),
        pltpu.VMEM((CHGMX, G_EDGE), _i32),
        pltpu.VMEM((G_EDGE, HID), _f32),
        pltpu.VMEM((G_EDGE, HID), _f32),
        pltpu.VMEM((RPT, HID), _f32),
        pltpu.VMEM((RPT, HID), _f32),
        pltpu.VMEM((RPT, HID), _f32),
        pltpu.VMEM((RPT,), _f32),
        pltpu.VMEM((16,), _f32),
        pltpu.VMEM_SHARED((NPAD, HID), _f32),
        pltpu.SemaphoreType.DMA,
        pltpu.SemaphoreType.DMA,
        pltpu.SemaphoreType.DMA,
        pltpu.SemaphoreType.DMA,
        pltpu.SemaphoreType.DMA,
    ],
)
def _sc_b(p_hbm, t1_hbm, norm_hbm, b1_hbm, dstf_hbm, srcf_hbm,
          out_q, out_t2, dd_v, src_v, rows0_v, rows1_v, t2_v, p1_v, t1_v, nv,
          b1_v, acc_sh, sg0, sg1, ss0, ss1, sh):
    cid = lax.axis_index("c")
    sid = lax.axis_index("s")

    pltpu.async_copy(p_hbm.at[0].at[pl.ds(sid * RPT, RPT)], t2_v, sh)
    _load_idx(dstf_hbm, srcf_hbm, dd_v, src_v, cid, sid)
    pltpu.sync_copy(p_hbm.at[1].at[pl.ds(sid * RPT, RPT)], p1_v)
    pltpu.sync_copy(t1_hbm.at[cid].at[pl.ds(sid * RPT, RPT)], t1_v)
    pltpu.sync_copy(norm_hbm.at[cid].at[pl.ds(sid * RPT, RPT)], nv)
    pltpu.sync_copy(b1_hbm, b1_v)
    _zero_vmem_rows(rows0_v, B_EDGE)
    for k in range(RPT // B_EDGE):
        pltpu.sync_copy(
            rows0_v.at[pl.ds(0, B_EDGE)],
            acc_sh.at[pl.ds(sid * RPT + k * B_EDGE, B_EDGE)],
        )
    pltpu.make_async_copy(p_hbm.at[0].at[pl.ds(sid * RPT, RPT)], t2_v, sh).wait()
    b1r = b1_v[...]

    def relu_body(i, carry):
        nb = _bcast_row(nv, i)
        agg = t2_v[i] + p1_v[i] + t1_v[i]
        h = jnp.maximum(agg * nb + b1r, 0.0)
        t2_v[i] = h * nb
        return carry

    lax.fori_loop(0, RPT, relu_body, 0)
    pltpu.sync_copy(t2_v, out_t2.at[cid].at[pl.ds(sid * RPT, RPT)])
    plsc.subcore_barrier()

    _split_agg(out_t2.at[cid], src_v, dd_v, rows0_v, rows1_v, acc_sh, cid,
               sg0, sg1, ss0, ss1)
    plsc.subcore_barrier()
    pltpu.sync_copy(
        acc_sh.at[pl.ds(sid * RPT, RPT)],
        out_q.at[cid].at[pl.ds(sid * RPT, RPT)],
    )


_RB = 1280
_RB2 = 1000


def _mm1_body(x_ref, w1_ref, o_ref):
    o_ref[...] = jnp.dot(x_ref[...], w1_ref[...], preferred_element_type=_f32)


def _post_body(qa_ref, qb_ref, t2_ref, norm_ref, w2_ref, b2_ref, o_ref):
    g = (qa_ref[0] + qb_ref[0] + t2_ref[0]) * norm_ref[...]
    z = jnp.dot(g, w2_ref[...], preferred_element_type=_f32) + b2_ref[...]
    m = jnp.max(z, axis=1, keepdims=True)
    e = z - m
    lse = jnp.log(jnp.sum(jnp.exp(e), axis=1, keepdims=True))
    o_ref[...] = e - lse


def _row_spec(c):
    return pl.BlockSpec((_RB, c), lambda i: (i, 0))


def _full_spec(r, c):
    return pl.BlockSpec((r, c), lambda i: (0, 0))


def kernel(x, edge_index, W1, b1, W2, b2):
    ep = jnp.pad(edge_index, ((0, 0), (0, E_PAD - N_EDGES)),
                 constant_values=N_NODES)
    srcf = ep[0].reshape(NS, 2 * CHG, G_EDGE)
    dstf = ep[1].reshape(NS, CHD, B_EDGE)
    b2r = b2.reshape(1, N_CLS)

    hw1 = pl.pallas_call(
        _mm1_body,
        grid=(N_NODES // _RB2,),
        in_specs=[pl.BlockSpec((_RB2, F_IN), lambda i: (i, 0)),
                  _full_spec(F_IN, HID)],
        out_specs=pl.BlockSpec((_RB2, HID), lambda i: (i, 0)),
        out_shape=jax.ShapeDtypeStruct((N_NODES, HID), _f32),
    )(x, W1)

    p, t1, norm, normc = _sc_a(hw1, dstf, srcf)

    q, t2 = _sc_b(p, t1, norm, b1, dstf, srcf)

    def _rs2(c):
        return pl.BlockSpec((_RB2, c), lambda i: (i, 0))

    def _rs3(lead):
        return pl.BlockSpec((1, _RB2, HID), lambda i, _l=lead: (_l, i, 0))

    out = pl.pallas_call(
        _post_body,
        grid=(N_NODES // _RB2,),
        in_specs=[_rs3(0), _rs3(1), _rs3(0), _rs2(1),
                  _full_spec(HID, N_CLS), _full_spec(1, N_CLS)],
        out_specs=_rs2(N_CLS),
        out_shape=jax.ShapeDtypeStruct((N_NODES, N_CLS), _f32),
    )(q, q, t2, normc.reshape(NPAD, 1), W2, b2r)

    return out


# speedup vs baseline: 1.0763x; 1.0032x over previous
import functools
import jax
import jax.numpy as jnp
from jax import lax
from jax.experimental import pallas as pl
from jax.experimental.pallas import tpu as pltpu
from jax.experimental.pallas import tpu_sc as plsc

N_NODES = 10000
N_EDGES = 320000
F_IN = 128
HID = 16
N_CLS = 100

NC = 2
NS = 16
NW = NC * NS
B_EDGE = 128
CH = 80
G_EDGE = 1024
CHG = (CH * B_EDGE) // G_EDGE
SPG = G_EDGE // B_EDGE
E_PAD = NW * CH * B_EDGE
CHD = 2 * CH
CHG0 = 14
CHG1 = 2 * CHG - CHG0
CH0 = CHG0 * SPG
CH1 = CHD - CH0
CHGMX = max(CHG0, CHG1)
NPAD = 10240
RPT = NPAD // NS
RPT_LAST = N_NODES - (NS - 1) * RPT

_mesh = plsc.VectorSubcoreMesh(
    core_axis_name="c", subcore_axis_name="s", num_cores=NC, num_subcores=NS
)
_sc_params = pltpu.CompilerParams(
    use_tc_tiling_on_sc=False, needs_layout_passes=False
)

_f32 = jnp.float32
_i32 = jnp.int32


def _bcast_row(vref, i):
    return plsc.load_gather(vref, [jnp.full((16,), i, _i32)])


def _row(ref2d, i):
    return plsc.load_gather(ref2d, [jnp.full((16,), i, _i32), lax.iota(_i32, 16)])


def _zero_vmem_rows(ref2d, n):
    for i in range(n):
        ref2d[i] = jnp.zeros((16,), _f32)


def _newton_rsqrt(x):
    i = plsc.bitcast(x, _i32)
    i = jnp.int32(0x5F3759DF) - lax.shift_right_arithmetic(i, 1)
    y = plsc.bitcast(i, _f32)
    for _ in range(3):
        y = y * (1.5 - 0.5 * x * y * y)
    return y


def _agg_pipeline(t_hbm, src_v, dd_v, rows0_v, rows1_v, acc_sh, base, chg,
                  sg0, sg1, ss0, ss1):
    def _scat_start(rows_v, chunk, sem):
        pltpu.async_copy(
            rows_v.at[pl.ds((chunk % SPG) * B_EDGE, B_EDGE)],
            acc_sh.at[dd_v.at[base + chunk]],
            sem,
            add=True,
        )

    def _scat(rows_v, chunk, sem):
        return pltpu.make_async_copy(
            rows_v.at[pl.ds((chunk % SPG) * B_EDGE, B_EDGE)],
            acc_sh.at[dd_v.at[base + chunk]],
            sem,
        )

    pltpu.async_copy(t_hbm.at[src_v.at[0]], rows0_v, sg0)

    def pair(i, carry):
        j0 = 2 * i
        j1 = j0 + 1
        pltpu.make_async_copy(t_hbm.at[src_v.at[j0]], rows0_v, sg0).wait()

        @pl.when(i > 0)
        def _():
            for k in range(SPG):
                _scat(rows1_v, SPG * (j0 - 1) + k, ss1).wait()

        pltpu.async_copy(t_hbm.at[src_v.at[j1]], rows1_v, sg1)
        for k in range(SPG):
            _scat_start(rows0_v, SPG * j0 + k, ss0)
        pltpu.make_async_copy(t_hbm.at[src_v.at[j1]], rows1_v, sg1).wait()
        for k in range(SPG):
            _scat(rows0_v, SPG * j0 + k, ss0).wait()

        @pl.when(i < chg // 2 - 1)
        def _():
            pltpu.async_copy(t_hbm.at[src_v.at[j0 + 2]], rows0_v, sg0)

        for k in range(SPG):
            _scat_start(rows1_v, SPG * j1 + k, ss1)
        return carry

    lax.fori_loop(0, chg // 2, pair, 0)
    for k in range(SPG):
        _scat(rows1_v, SPG * (chg - 1) + k, ss1).wait()


def _split_agg(t_hbm, src_v, dd_v, rows0_v, rows1_v, acc_sh, cid,
               sg0, sg1, ss0, ss1):
    @pl.when(cid == 0)
    def _():
        _agg_pipeline(t_hbm, src_v, dd_v, rows0_v, rows1_v, acc_sh, 0, CHG0,
                      sg0, sg1, ss0, ss1)

    @pl.when(cid == 1)
    def _():
        _agg_pipeline(t_hbm, src_v, dd_v, rows0_v, rows1_v, acc_sh, CH0, CHG1,
                      sg0, sg1, ss0, ss1)


def _load_idx(dstf_hbm, srcf_hbm, dd_v, src_v, cid, sid):
    pltpu.sync_copy(dstf_hbm.at[sid], dd_v)

    @pl.when(cid == 0)
    def _():
        pltpu.sync_copy(srcf_hbm.at[sid].at[pl.ds(0, CHG0)],
                        src_v.at[pl.ds(0, CHG0)])

    @pl.when(cid == 1)
    def _():
        pltpu.sync_copy(srcf_hbm.at[sid].at[pl.ds(CHG0, CHG1)],
                        src_v.at[pl.ds(0, CHG1)])


@functools.partial(
    pl.kernel,
    out_type=(
        jax.ShapeDtypeStruct((NC, NPAD, HID), _f32),
        jax.ShapeDtypeStruct((NC, NPAD, HID), _f32),
        jax.ShapeDtypeStruct((NC, NPAD), _f32),
        jax.ShapeDtypeStruct((NPAD,), _f32),
    ),
    mesh=_mesh,
    compiler_params=_sc_params,
    scratch_types=[
        pltpu.VMEM((CHD, B_EDGE), _i32),
        pltpu.VMEM((CHGMX, G_EDGE), _i32),
        pltpu.VMEM((G_EDGE, HID), _f32),
        pltpu.VMEM((G_EDGE, HID), _f32),
        pltpu.VMEM((RPT, HID), _f32),
        pltpu.VMEM((RPT,), _f32),
        pltpu.VMEM((B_EDGE,), _f32),
        pltpu.VMEM_SHARED((NPAD, HID), _f32),
        pltpu.VMEM_SHARED((NPAD,), _f32),
        pltpu.SemaphoreType.DMA,
        pltpu.SemaphoreType.DMA,
        pltpu.SemaphoreType.DMA,
        pltpu.SemaphoreType.DMA,
        pltpu.SemaphoreType.DMA,
    ],
)
def _sc_a(hw1_hbm, dstf_hbm, srcf_hbm,
          out_p, out_t1, out_norm, out_normc,
          dd_v, src_v, rows0_v, rows1_v, t1_v, nv, ones_v,
          acc_sh, accd_sh, sg0, sg1, ss0, ss1, sh):
    cid = lax.axis_index("c")
    sid = lax.axis_index("s")

    @pl.when(sid < NS - 1)
    def _():
        pltpu.async_copy(hw1_hbm.at[pl.ds(sid * RPT, RPT)], t1_v, sh)

    @pl.when(sid == NS - 1)
    def _():
        pltpu.async_copy(
            hw1_hbm.at[pl.ds((NS - 1) * RPT, RPT_LAST)],
            t1_v.at[pl.ds(0, RPT_LAST)], sh,
        )
        for i in range(RPT_LAST, RPT):
            t1_v[i] = jnp.zeros((16,), _f32)

    _load_idx(dstf_hbm, srcf_hbm, dd_v, src_v, cid, sid)

    for i in range(RPT // 16):
        nv[pl.ds(i * 16, 16)] = jnp.zeros((16,), _f32)
    pltpu.sync_copy(nv, accd_sh.at[pl.ds(sid * RPT, RPT)])
    @pl.when(cid == 1)
    def _():
        _zero_vmem_rows(rows0_v, B_EDGE)
        for k in range(RPT // B_EDGE):
            pltpu.sync_copy(
                rows0_v.at[pl.ds(0, B_EDGE)],
                acc_sh.at[pl.ds(sid * RPT + k * B_EDGE, B_EDGE)],
            )

    for i in range(B_EDGE // 16):
        ones_v[pl.ds(i * 16, 16)] = jnp.ones((16,), _f32)
    plsc.subcore_barrier()

    def deg_body(i, carry):
        for k in range(8):
            pltpu.async_copy(
                ones_v, accd_sh.at[dd_v.at[8 * i + k]], ss0, add=True
            )
        for k in range(8):
            pltpu.make_async_copy(
                ones_v, accd_sh.at[dd_v.at[8 * i + k]], ss0
            ).wait()
        return carry

    lax.fori_loop(0, CHD // 8, deg_body, 0)
    plsc.subcore_barrier()

    pltpu.sync_copy(accd_sh.at[pl.ds(sid * RPT, RPT)], nv)
    for i in range(RPT // 16):
        d = nv[pl.ds(i * 16, 16)]
        nv[pl.ds(i * 16, 16)] = _newton_rsqrt(d + 1.0)
    pltpu.sync_copy(nv, out_norm.at[cid].at[pl.ds(sid * RPT, RPT)])

    @pl.when(cid == 0)
    def _():
        pltpu.sync_copy(nv, out_normc.at[pl.ds(sid * RPT, RPT)])

    @pl.when(sid < NS - 1)
    def _():
        pltpu.make_async_copy(
            hw1_hbm.at[pl.ds(sid * RPT, RPT)], t1_v, sh
        ).wait()

    @pl.when(sid == NS - 1)
    def _():
        pltpu.make_async_copy(
            hw1_hbm.at[pl.ds((NS - 1) * RPT, RPT_LAST)],
            t1_v.at[pl.ds(0, RPT_LAST)], sh,
        ).wait()

    def scale_body(i, carry):
        t1_v[i] = t1_v[i] * _bcast_row(nv, i)
        return carry

    lax.fori_loop(0, RPT, scale_body, 0)
    pltpu.sync_copy(t1_v, out_t1.at[cid].at[pl.ds(sid * RPT, RPT)])

    @pl.when(cid == 0)
    def _():
        pltpu.sync_copy(t1_v, acc_sh.at[pl.ds(sid * RPT, RPT)])

    plsc.subcore_barrier()

    _split_agg(out_t1.at[cid], src_v, dd_v, rows0_v, rows1_v, acc_sh, cid,
               sg0, sg1, ss0, ss1)
    plsc.subcore_barrier()
    pltpu.sync_copy(
        acc_sh.at[pl.ds(sid * RPT, RPT)],
        out_p.at[cid].at[pl.ds(sid * RPT, RPT)],
    )


@functools.partial(
    pl.kernel,
    out_type=(
        jax.ShapeDtypeStruct((NC, NPAD, HID), _f32),
        jax.ShapeDtypeStruct((NC, NPAD, HID), _f32),
    ),
    mesh=_mesh,
    compiler_params=_sc_params,
    scratch_types=[
        pltpu.VMEM((CHD, B_EDGE), _i32),
        pltpu.VMEM((CHGMX, G_EDGE), _i32),
        pltpu.VMEM((G_EDGE, HID), _f32),
        pltpu.VMEM((G_EDGE, HID), _f32),
        pltpu.VMEM((RPT, HID), _f32),
        pltpu.VMEM((RPT, HID), _f32),
        pltpu.VMEM((RPT,), _f32),
        pltpu.VMEM((16,), _f32),
        pltpu.VMEM_SHARED((NPAD, HID), _f32),
        pltpu.SemaphoreType.DMA,
        pltpu.SemaphoreType.DMA,
        pltpu.SemaphoreType.DMA,
        pltpu.SemaphoreType.DMA,
        pltpu.SemaphoreType.DMA,
    ],
)
def _sc_b(p_hbm, norm_hbm, b1_hbm, dstf_hbm, srcf_hbm,
          out_q, out_t2, dd_v, src_v, rows0_v, rows1_v, t2_v, p1_v, nv,
          b1_v, acc_sh, sg0, sg1, ss0, ss1, sh):
    cid = lax.axis_index("c")
    sid = lax.axis_index("s")

    pltpu.async_copy(p_hbm.at[0].at[pl.ds(sid * RPT, RPT)], t2_v, sh)
    _load_idx(dstf_hbm, srcf_hbm, dd_v, src_v, cid, sid)
    pltpu.sync_copy(p_hbm.at[1].at[pl.ds(sid * RPT, RPT)], p1_v)
    pltpu.sync_copy(norm_hbm.at[cid].at[pl.ds(sid * RPT, RPT)], nv)
    pltpu.sync_copy(b1_hbm, b1_v)

    @pl.when(cid == 1)
    def _():
        _zero_vmem_rows(rows0_v, B_EDGE)
        for k in range(RPT // B_EDGE):
            pltpu.sync_copy(
                rows0_v.at[pl.ds(0, B_EDGE)],
                acc_sh.at[pl.ds(sid * RPT + k * B_EDGE, B_EDGE)],
            )

    pltpu.make_async_copy(p_hbm.at[0].at[pl.ds(sid * RPT, RPT)], t2_v, sh).wait()
    b1r = b1_v[...]

    def relu_body(i, carry):
        nb = _bcast_row(nv, i)
        agg = t2_v[i] + p1_v[i]
        h = jnp.maximum(agg * nb + b1r, 0.0)
        t2_v[i] = h * nb
        return carry

    lax.fori_loop(0, RPT, relu_body, 0)
    pltpu.sync_copy(t2_v, out_t2.at[cid].at[pl.ds(sid * RPT, RPT)])

    @pl.when(cid == 0)
    def _():
        pltpu.sync_copy(t2_v, acc_sh.at[pl.ds(sid * RPT, RPT)])

    plsc.subcore_barrier()

    _split_agg(out_t2.at[cid], src_v, dd_v, rows0_v, rows1_v, acc_sh, cid,
               sg0, sg1, ss0, ss1)
    plsc.subcore_barrier()
    pltpu.sync_copy(
        acc_sh.at[pl.ds(sid * RPT, RPT)],
        out_q.at[cid].at[pl.ds(sid * RPT, RPT)],
    )


_RB = 1280
_RB2 = 1000


def _mm1_body(x_ref, w1_ref, o_ref):
    o_ref[...] = jnp.dot(x_ref[...], w1_ref[...], preferred_element_type=_f32)


def _post_body(qa_ref, qb_ref, norm_ref, w2_ref, b2_ref, o_ref):
    g = (qa_ref[0] + qb_ref[0]) * norm_ref[...]
    z = jnp.dot(g, w2_ref[...], preferred_element_type=_f32) + b2_ref[...]
    m = jnp.max(z, axis=1, keepdims=True)
    e = z - m
    lse = jnp.log(jnp.sum(jnp.exp(e), axis=1, keepdims=True))
    o_ref[...] = e - lse


def _row_spec(c):
    return pl.BlockSpec((_RB, c), lambda i: (i, 0))


def _full_spec(r, c):
    return pl.BlockSpec((r, c), lambda i: (0, 0))


def kernel(x, edge_index, W1, b1, W2, b2):
    ep = jnp.pad(edge_index, ((0, 0), (0, E_PAD - N_EDGES)),
                 constant_values=N_NODES)
    srcf = ep[0].reshape(NS, 2 * CHG, G_EDGE)
    dstf = ep[1].reshape(NS, CHD, B_EDGE)
    b2r = b2.reshape(1, N_CLS)

    hw1 = pl.pallas_call(
        _mm1_body,
        grid=(N_NODES // _RB2,),
        in_specs=[pl.BlockSpec((_RB2, F_IN), lambda i: (i, 0)),
                  _full_spec(F_IN, HID)],
        out_specs=pl.BlockSpec((_RB2, HID), lambda i: (i, 0)),
        out_shape=jax.ShapeDtypeStruct((N_NODES, HID), _f32),
    )(x, W1)

    p, t1, norm, normc = _sc_a(hw1, dstf, srcf)

    q, t2 = _sc_b(p, norm, b1, dstf, srcf)

    def _rs2(c):
        return pl.BlockSpec((_RB2, c), lambda i: (i, 0))

    def _rs3(lead):
        return pl.BlockSpec((1, _RB2, HID), lambda i, _l=lead: (_l, i, 0))

    out = pl.pallas_call(
        _post_body,
        grid=(N_NODES // _RB2,),
        in_specs=[_rs3(0), _rs3(1), _rs2(1),
                  _full_spec(HID, N_CLS), _full_spec(1, N_CLS)],
        out_specs=_rs2(N_CLS),
        out_shape=jax.ShapeDtypeStruct((N_NODES, N_CLS), _f32),
    )(q, q, normc.reshape(NPAD, 1), W2, b2r)

    return out


# speedup vs baseline: 1.0872x; 1.0101x over previous
import functools
import jax
import jax.numpy as jnp
from jax import lax
from jax.experimental import pallas as pl
from jax.experimental.pallas import tpu as pltpu
from jax.experimental.pallas import tpu_sc as plsc

N_NODES = 10000
N_EDGES = 320000
F_IN = 128
HID = 16
N_CLS = 100

NC = 2
NS = 16
NW = NC * NS
B_EDGE = 128
CH = 80
G_EDGE = 1024
CHG = (CH * B_EDGE) // G_EDGE
SPG = G_EDGE // B_EDGE
E_PAD = NW * CH * B_EDGE
CHD = 2 * CH
CHG0 = 14
CHG1 = 2 * CHG - CHG0
CH0 = CHG0 * SPG
CH1 = CHD - CH0
CHGMX = max(CHG0, CHG1)
NPAD = 10240
RPT = NPAD // NS
RPT_LAST = N_NODES - (NS - 1) * RPT

_mesh = plsc.VectorSubcoreMesh(
    core_axis_name="c", subcore_axis_name="s", num_cores=NC, num_subcores=NS
)
_sc_params = pltpu.CompilerParams(
    use_tc_tiling_on_sc=False, needs_layout_passes=False
)

_f32 = jnp.float32
_i32 = jnp.int32


def _bcast_row(vref, i):
    return plsc.load_gather(vref, [jnp.full((16,), i, _i32)])


def _row(ref2d, i):
    return plsc.load_gather(ref2d, [jnp.full((16,), i, _i32), lax.iota(_i32, 16)])


def _zero_vmem_rows(ref2d, n):
    for i in range(n):
        ref2d[i] = jnp.zeros((16,), _f32)


def _newton_rsqrt(x):
    i = plsc.bitcast(x, _i32)
    i = jnp.int32(0x5F3759DF) - lax.shift_right_arithmetic(i, 1)
    y = plsc.bitcast(i, _f32)
    for _ in range(3):
        y = y * (1.5 - 0.5 * x * y * y)
    return y


def _agg_pipeline(t_hbm, src_v, dd_v, rows0_v, rows1_v, acc_sh, base, chg,
                  sg0, sg1, ss0, ss1):
    def _scat_start(rows_v, chunk, sem):
        pltpu.async_copy(
            rows_v.at[pl.ds((chunk % SPG) * B_EDGE, B_EDGE)],
            acc_sh.at[dd_v.at[base + chunk]],
            sem,
            add=True,
        )

    def _scat(rows_v, chunk, sem):
        return pltpu.make_async_copy(
            rows_v.at[pl.ds((chunk % SPG) * B_EDGE, B_EDGE)],
            acc_sh.at[dd_v.at[base + chunk]],
            sem,
        )

    pltpu.async_copy(t_hbm.at[src_v.at[0]], rows0_v, sg0)

    def pair(i, carry):
        j0 = 2 * i
        j1 = j0 + 1
        pltpu.make_async_copy(t_hbm.at[src_v.at[j0]], rows0_v, sg0).wait()

        @pl.when(i > 0)
        def _():
            for k in range(SPG):
                _scat(rows1_v, SPG * (j0 - 1) + k, ss1).wait()

        pltpu.async_copy(t_hbm.at[src_v.at[j1]], rows1_v, sg1)
        for k in range(SPG):
            _scat_start(rows0_v, SPG * j0 + k, ss0)
        pltpu.make_async_copy(t_hbm.at[src_v.at[j1]], rows1_v, sg1).wait()
        for k in range(SPG):
            _scat_start(rows1_v, SPG * j1 + k, ss1)
        for k in range(SPG):
            _scat(rows0_v, SPG * j0 + k, ss0).wait()

        @pl.when(i < chg // 2 - 1)
        def _():
            pltpu.async_copy(t_hbm.at[src_v.at[j0 + 2]], rows0_v, sg0)

        return carry

    lax.fori_loop(0, chg // 2, pair, 0)
    for k in range(SPG):
        _scat(rows1_v, SPG * (chg - 1) + k, ss1).wait()


def _split_agg(t_hbm, src_v, dd_v, rows0_v, rows1_v, acc_sh, cid,
               sg0, sg1, ss0, ss1):
    @pl.when(cid == 0)
    def _():
        _agg_pipeline(t_hbm, src_v, dd_v, rows0_v, rows1_v, acc_sh, 0, CHG0,
                      sg0, sg1, ss0, ss1)

    @pl.when(cid == 1)
    def _():
        _agg_pipeline(t_hbm, src_v, dd_v, rows0_v, rows1_v, acc_sh, CH0, CHG1,
                      sg0, sg1, ss0, ss1)


def _load_idx(dstf_hbm, srcf_hbm, dd_v, src_v, cid, sid):
    pltpu.sync_copy(dstf_hbm.at[sid], dd_v)

    @pl.when(cid == 0)
    def _():
        pltpu.sync_copy(srcf_hbm.at[sid].at[pl.ds(0, CHG0)],
                        src_v.at[pl.ds(0, CHG0)])

    @pl.when(cid == 1)
    def _():
        pltpu.sync_copy(srcf_hbm.at[sid].at[pl.ds(CHG0, CHG1)],
                        src_v.at[pl.ds(0, CHG1)])


@functools.partial(
    pl.kernel,
    out_type=(
        jax.ShapeDtypeStruct((NC, NPAD, HID), _f32),
        jax.ShapeDtypeStruct((NC, NPAD, HID), _f32),
        jax.ShapeDtypeStruct((NC, NPAD), _f32),
        jax.ShapeDtypeStruct((NPAD,), _f32),
    ),
    mesh=_mesh,
    compiler_params=_sc_params,
    scratch_types=[
        pltpu.VMEM((CHD, B_EDGE), _i32),
        pltpu.VMEM((CHGMX, G_EDGE), _i32),
        pltpu.VMEM((G_EDGE, HID), _f32),
        pltpu.VMEM((G_EDGE, HID), _f32),
        pltpu.VMEM((RPT, HID), _f32),
        pltpu.VMEM((RPT,), _f32),
        pltpu.VMEM((B_EDGE,), _f32),
        pltpu.VMEM_SHARED((NPAD, HID), _f32),
        pltpu.VMEM_SHARED((NPAD,), _f32),
        pltpu.SemaphoreType.DMA,
        pltpu.SemaphoreType.DMA,
        pltpu.SemaphoreType.DMA,
        pltpu.SemaphoreType.DMA,
        pltpu.SemaphoreType.DMA,
    ],
)
def _sc_a(hw1_hbm, dstf_hbm, srcf_hbm,
          out_p, out_t1, out_norm, out_normc,
          dd_v, src_v, rows0_v, rows1_v, t1_v, nv, ones_v,
          acc_sh, accd_sh, sg0, sg1, ss0, ss1, sh):
    cid = lax.axis_index("c")
    sid = lax.axis_index("s")

    @pl.when(sid < NS - 1)
    def _():
        pltpu.async_copy(hw1_hbm.at[pl.ds(sid * RPT, RPT)], t1_v, sh)

    @pl.when(sid == NS - 1)
    def _():
        pltpu.async_copy(
            hw1_hbm.at[pl.ds((NS - 1) * RPT, RPT_LAST)],
            t1_v.at[pl.ds(0, RPT_LAST)], sh,
        )
        for i in range(RPT_LAST, RPT):
            t1_v[i] = jnp.zeros((16,), _f32)

    _load_idx(dstf_hbm, srcf_hbm, dd_v, src_v, cid, sid)

    for i in range(RPT // 16):
        nv[pl.ds(i * 16, 16)] = jnp.zeros((16,), _f32)
    pltpu.sync_copy(nv, accd_sh.at[pl.ds(sid * RPT, RPT)])
    @pl.when(cid == 1)
    def _():
        _zero_vmem_rows(rows0_v, B_EDGE)
        for k in range(RPT // B_EDGE):
            pltpu.sync_copy(
                rows0_v.at[pl.ds(0, B_EDGE)],
                acc_sh.at[pl.ds(sid * RPT + k * B_EDGE, B_EDGE)],
            )

    for i in range(B_EDGE // 16):
        ones_v[pl.ds(i * 16, 16)] = jnp.ones((16,), _f32)
    plsc.subcore_barrier()

    def deg_body(i, carry):
        for k in range(16):
            pltpu.async_copy(
                ones_v, accd_sh.at[dd_v.at[16 * i + k]], ss0, add=True
            )
        for k in range(16):
            pltpu.make_async_copy(
                ones_v, accd_sh.at[dd_v.at[16 * i + k]], ss0
            ).wait()
        return carry

    lax.fori_loop(0, CHD // 16, deg_body, 0)
    plsc.subcore_barrier()

    pltpu.sync_copy(accd_sh.at[pl.ds(sid * RPT, RPT)], nv)
    for i in range(RPT // 16):
        d = nv[pl.ds(i * 16, 16)]
        nv[pl.ds(i * 16, 16)] = _newton_rsqrt(d + 1.0)
    pltpu.sync_copy(nv, out_norm.at[cid].at[pl.ds(sid * RPT, RPT)])

    @pl.when(cid == 0)
    def _():
        pltpu.sync_copy(nv, out_normc.at[pl.ds(sid * RPT, RPT)])

    @pl.when(sid < NS - 1)
    def _():
        pltpu.make_async_copy(
            hw1_hbm.at[pl.ds(sid * RPT, RPT)], t1_v, sh
        ).wait()

    @pl.when(sid == NS - 1)
    def _():
        pltpu.make_async_copy(
            hw1_hbm.at[pl.ds((NS - 1) * RPT, RPT_LAST)],
            t1_v.at[pl.ds(0, RPT_LAST)], sh,
        ).wait()

    def scale_body(i, carry):
        for u in range(4):
            r = 4 * i + u
            t1_v[r] = t1_v[r] * _bcast_row(nv, r)
        return carry

    lax.fori_loop(0, RPT // 4, scale_body, 0)
    pltpu.sync_copy(t1_v, out_t1.at[cid].at[pl.ds(sid * RPT, RPT)])

    @pl.when(cid == 0)
    def _():
        pltpu.sync_copy(t1_v, acc_sh.at[pl.ds(sid * RPT, RPT)])

    plsc.subcore_barrier()

    _split_agg(out_t1.at[cid], src_v, dd_v, rows0_v, rows1_v, acc_sh, cid,
               sg0, sg1, ss0, ss1)
    plsc.subcore_barrier()
    pltpu.sync_copy(
        acc_sh.at[pl.ds(sid * RPT, RPT)],
        out_p.at[cid].at[pl.ds(sid * RPT, RPT)],
    )


@functools.partial(
    pl.kernel,
    out_type=(
        jax.ShapeDtypeStruct((NC, NPAD, HID), _f32),
        jax.ShapeDtypeStruct((NC, NPAD, HID), _f32),
    ),
    mesh=_mesh,
    compiler_params=_sc_params,
    scratch_types=[
        pltpu.VMEM((CHD, B_EDGE), _i32),
        pltpu.VMEM((CHGMX, G_EDGE), _i32),
        pltpu.VMEM((G_EDGE, HID), _f32),
        pltpu.VMEM((G_EDGE, HID), _f32),
        pltpu.VMEM((RPT, HID), _f32),
        pltpu.VMEM((RPT, HID), _f32),
        pltpu.VMEM((RPT,), _f32),
        pltpu.VMEM((16,), _f32),
        pltpu.VMEM_SHARED((NPAD, HID), _f32),
        pltpu.SemaphoreType.DMA,
        pltpu.SemaphoreType.DMA,
        pltpu.SemaphoreType.DMA,
        pltpu.SemaphoreType.DMA,
        pltpu.SemaphoreType.DMA,
    ],
)
def _sc_b(p_hbm, norm_hbm, b1_hbm, dstf_hbm, srcf_hbm,
          out_q, out_t2, dd_v, src_v, rows0_v, rows1_v, t2_v, p1_v, nv,
          b1_v, acc_sh, sg0, sg1, ss0, ss1, sh):
    cid = lax.axis_index("c")
    sid = lax.axis_index("s")

    pltpu.async_copy(p_hbm.at[0].at[pl.ds(sid * RPT, RPT)], t2_v, sh)
    _load_idx(dstf_hbm, srcf_hbm, dd_v, src_v, cid, sid)
    pltpu.sync_copy(p_hbm.at[1].at[pl.ds(sid * RPT, RPT)], p1_v)
    pltpu.sync_copy(norm_hbm.at[cid].at[pl.ds(sid * RPT, RPT)], nv)
    pltpu.sync_copy(b1_hbm, b1_v)

    @pl.when(cid == 1)
    def _():
        _zero_vmem_rows(rows0_v, B_EDGE)
        for k in range(RPT // B_EDGE):
            pltpu.sync_copy(
                rows0_v.at[pl.ds(0, B_EDGE)],
                acc_sh.at[pl.ds(sid * RPT + k * B_EDGE, B_EDGE)],
            )

    pltpu.make_async_copy(p_hbm.at[0].at[pl.ds(sid * RPT, RPT)], t2_v, sh).wait()
    b1r = b1_v[...]

    def relu_body(i, carry):
        for u in range(4):
            r = 4 * i + u
            nb = _bcast_row(nv, r)
            h = jnp.maximum((t2_v[r] + p1_v[r]) * nb + b1r, 0.0)
            t2_v[r] = h * nb
        return carry

    lax.fori_loop(0, RPT // 4, relu_body, 0)
    pltpu.sync_copy(t2_v, out_t2.at[cid].at[pl.ds(sid * RPT, RPT)])

    @pl.when(cid == 0)
    def _():
        pltpu.sync_copy(t2_v, acc_sh.at[pl.ds(sid * RPT, RPT)])

    plsc.subcore_barrier()

    _split_agg(out_t2.at[cid], src_v, dd_v, rows0_v, rows1_v, acc_sh, cid,
               sg0, sg1, ss0, ss1)
    plsc.subcore_barrier()
    pltpu.sync_copy(
        acc_sh.at[pl.ds(sid * RPT, RPT)],
        out_q.at[cid].at[pl.ds(sid * RPT, RPT)],
    )


_RB = 1280
_RB2 = 1000


def _mm1_body(x_ref, w1_ref, o_ref):
    o_ref[...] = jnp.dot(x_ref[...], w1_ref[...], preferred_element_type=_f32)


def _post_body(qa_ref, qb_ref, norm_ref, w2_ref, b2_ref, o_ref):
    g = (qa_ref[0] + qb_ref[0]) * norm_ref[...]
    z = jnp.dot(g, w2_ref[...], preferred_element_type=_f32) + b2_ref[...]
    m = jnp.max(z, axis=1, keepdims=True)
    e = z - m
    lse = jnp.log(jnp.sum(jnp.exp(e), axis=1, keepdims=True))
    o_ref[...] = e - lse


def _row_spec(c):
    return pl.BlockSpec((_RB, c), lambda i: (i, 0))


def _full_spec(r, c):
    return pl.BlockSpec((r, c), lambda i: (0, 0))


def kernel(x, edge_index, W1, b1, W2, b2):
    ep = jnp.pad(edge_index, ((0, 0), (0, E_PAD - N_EDGES)),
                 constant_values=N_NODES)
    srcf = ep[0].reshape(NS, 2 * CHG, G_EDGE)
    dstf = ep[1].reshape(NS, CHD, B_EDGE)
    b2r = b2.reshape(1, N_CLS)

    hw1 = pl.pallas_call(
        _mm1_body,
        grid=(N_NODES // _RB2,),
        in_specs=[pl.BlockSpec((_RB2, F_IN), lambda i: (i, 0)),
                  _full_spec(F_IN, HID)],
        out_specs=pl.BlockSpec((_RB2, HID), lambda i: (i, 0)),
        out_shape=jax.ShapeDtypeStruct((N_NODES, HID), _f32),
    )(x, W1)

    p, t1, norm, normc = _sc_a(hw1, dstf, srcf)

    q, t2 = _sc_b(p, norm, b1, dstf, srcf)

    def _rs2(c):
        return pl.BlockSpec((_RB2, c), lambda i: (i, 0))

    def _rs3(lead):
        return pl.BlockSpec((1, _RB2, HID), lambda i, _l=lead: (_l, i, 0))

    out = pl.pallas_call(
        _post_body,
        grid=(N_NODES // _RB2,),
        in_specs=[_rs3(0), _rs3(1), _rs2(1),
                  _full_spec(HID, N_CLS), _full_spec(1, N_CLS)],
        out_specs=_rs2(N_CLS),
        out_shape=jax.ShapeDtypeStruct((N_NODES, N_CLS), _f32),
    )(q, q, normc.reshape(NPAD, 1), W2, b2r)

    return out
